# Initial kernel scaffold; baseline (speedup 1.0000x reference)
import jax
import jax.numpy as jnp
from jax.experimental import pallas as pl


def kernel(x, table):
    raise NotImplementedError("write your pallas kernel here")


# speedup vs baseline: 1.5768x; 1.5768x over previous
import functools

import jax
import jax.numpy as jnp
from jax import lax
from jax.experimental import pallas as pl
from jax.experimental.pallas import tpu as pltpu
from jax.experimental.pallas import tpu_sc as plsc

V_NUM = 1000000
H_DIM = 32
BATCH = 16384
FIELDS = 26

_N = BATCH * FIELDS
_NW = 32
_PER_W = _N // _NW
_CH = 832
_NCH = _PER_W // _CH


def _make_kernel():
    mesh = plsc.VectorSubcoreMesh(core_axis_name="c", subcore_axis_name="s")

    @functools.partial(
        pl.kernel,
        mesh=mesh,
        out_type=jax.ShapeDtypeStruct((_N, H_DIM), jnp.float32),
        compiler_params=pltpu.CompilerParams(use_tc_tiling_on_sc=False),
        scratch_types=[
            pltpu.VMEM((_PER_W,), jnp.int32),
            pltpu.VMEM((_CH, H_DIM), jnp.float32),
            pltpu.VMEM((_CH, H_DIM), jnp.float32),
            pltpu.SemaphoreType.DMA,
            pltpu.SemaphoreType.DMA,
            pltpu.SemaphoreType.DMA,
            pltpu.SemaphoreType.DMA,
        ],
    )
    def gather_kernel(idx_hbm, table_hbm, out_hbm, idx_v, rows0, rows1,
                      gsem0, gsem1, osem0, osem1):
        wid = lax.axis_index("s") * 2 + lax.axis_index("c")
        base = wid * _PER_W
        pltpu.sync_copy(idx_hbm.at[pl.ds(base, _PER_W)], idx_v)

        rows = (rows0, rows1)
        gsem = (gsem0, gsem1)
        osem = (osem0, osem1)

        for b in range(2):
            pltpu.async_copy(
                table_hbm.at[idx_v.at[pl.ds(b * _CH, _CH)]], rows[b], gsem[b])

        for j in range(_NCH):
            b = j % 2
            pltpu.make_async_copy(
                table_hbm.at[idx_v.at[pl.ds(j * _CH, _CH)]], rows[b],
                gsem[b]).wait()
            out_slice = out_hbm.at[pl.ds(base + j * _CH, _CH)]
            pltpu.async_copy(rows[b], out_slice, osem[b])
            nxt = j + 2
            if nxt < _NCH:
                pltpu.make_async_copy(rows[b], out_slice, osem[b]).wait()
                pltpu.async_copy(
                    table_hbm.at[idx_v.at[pl.ds(nxt * _CH, _CH)]], rows[b],
                    gsem[b])

        for j in range(_NCH - 2, _NCH):
            b = j % 2
            pltpu.make_async_copy(
                rows[b], out_hbm.at[pl.ds(base + j * _CH, _CH)],
                osem[b]).wait()

    return gather_kernel


_KERNEL = _make_kernel()


@jax.jit
def kernel(x, table):
    idx = x.reshape(-1).astype(jnp.int32)
    out = _KERNEL(idx, table)
    return out.reshape(BATCH, FIELDS, H_DIM)
